# baseline (speedup 1.0000x reference)
import jax
import jax.numpy as jnp
from jax.experimental import pallas as pl
from jax.experimental.pallas import tpu as pltpu

B, L, V = 64, 4, 100000
R = B * L
CW = 4096
NC = (V + CW - 1) // CW
TAIL = V - (NC - 1) * CW

_KS0 = 0
_KS1 = 42
_KS2 = 0x1BD11BDA ^ _KS0 ^ _KS1

_NEG_INF = float("-inf")
_TINY = 1.1754943508222875e-38


def _rotl(x, r):
    return jax.lax.shift_left(x, r) | jax.lax.shift_right_logical(x, 32 - r)


def _threefry_bits(fk):
    v0 = jnp.zeros_like(fk) + jnp.int32(_KS0)
    v1 = fk

    def rounds(v0, v1, rots):
        for r in rots:
            v0 = v0 + v1
            v1 = _rotl(v1, r) ^ v0
        return v0, v1

    r0 = (13, 15, 26, 6)
    r1 = (17, 29, 16, 24)
    v0, v1 = rounds(v0, v1, r0)
    v0 += jnp.int32(_KS1); v1 += jnp.int32(_KS2 + 1)
    v0, v1 = rounds(v0, v1, r1)
    v0 += jnp.int32(_KS2); v1 += jnp.int32(_KS0 + 2)
    v0, v1 = rounds(v0, v1, r0)
    v0 += jnp.int32(_KS0); v1 += jnp.int32(_KS1 + 3)
    v0, v1 = rounds(v0, v1, r1)
    v0 += jnp.int32(_KS1); v1 += jnp.int32(_KS2 + 4)
    v0, v1 = rounds(v0, v1, r0)
    v0 += jnp.int32(_KS2); v1 += jnp.int32(_KS0 + 5)
    return v0 ^ v1


def _gumbel_y(f, x):
    bits = _threefry_bits(f)
    fb = jax.lax.shift_right_logical(bits, 9) | jnp.int32(0x3F800000)
    u = jax.lax.bitcast_convert_type(fb, jnp.float32) - jnp.float32(1.0)
    u = jnp.maximum(u, jnp.float32(_TINY))
    return -jnp.log(-jnp.log(u)) + x


def _kernel(x_ref, out_ref, ry_ref, rc_ref, fb_ref):
    pc = pl.program_id(0)

    @pl.when(pc == 0)
    def _first():
        row = jax.lax.broadcasted_iota(jnp.int32, (R, CW), 0)
        lane = jax.lax.broadcasted_iota(jnp.int32, (R, CW), 1)
        f0 = row * V + lane + _KS1
        fb_ref[...] = f0
        y = _gumbel_y(f0 + (NC - 1) * CW, x_ref[...])
        ry_ref[...] = jnp.where(lane < TAIL, y, _NEG_INF)
        rc_ref[...] = jnp.full((R, CW), NC - 1, jnp.int32)

    @pl.when(pc > 0)
    def _main():
        cid = pc - 1
        y = _gumbel_y(fb_ref[...] + cid * CW, x_ref[...])
        ry = ry_ref[...]
        upd = y > ry
        ry_ref[...] = jnp.where(upd, y, ry)
        rc_ref[...] = jnp.where(upd, cid, rc_ref[...])

    @pl.when(pc == NC - 1)
    def _finish():
        lane = jax.lax.broadcasted_iota(jnp.int32, (R, CW), 1)
        ry = ry_ref[...]
        col = rc_ref[...] * CW + lane
        m = jnp.max(ry, axis=1, keepdims=True)
        idx = jnp.min(jnp.where(ry == m, col, jnp.int32(V)), axis=1)
        out_ref[...] = idx.reshape(1, 1, R)


def kernel(logits):
    x = logits.reshape(R, V)
    out = pl.pallas_call(
        _kernel,
        grid=(NC,),
        in_specs=[pl.BlockSpec((R, CW), lambda c: (0, (c + NC - 1) % NC))],
        out_specs=pl.BlockSpec((1, 1, R), lambda c: (0, 0, 0)),
        out_shape=jax.ShapeDtypeStruct((1, 1, R), jnp.int32),
        scratch_shapes=[
            pltpu.VMEM((R, CW), jnp.float32),
            pltpu.VMEM((R, CW), jnp.int32),
            pltpu.VMEM((R, CW), jnp.int32),
        ],
    )(x)
    return out.reshape(B, L)
